# Initial kernel scaffold; baseline (speedup 1.0000x reference)
import functools

import jax
import jax.numpy as jnp
from jax import lax
from jax.experimental import pallas as pl
from jax.experimental.pallas import tpu as pltpu
from jax.experimental.pallas import tpu_sc as plsc

N_NODES = 10000
N_EDGES = 320000
D_IN = 128
D_HID = 256
D_OUT = 128

NC = 2
NS = 16
NW = NC * NS
CHUNK = 128
CPW = 79
E_PAD = NW * CPW * CHUNK
N_ACC = N_NODES + 16
ROWS_PER_SUB = N_ACC // NS


@functools.partial(
    pl.kernel,
    out_type=(
        jax.ShapeDtypeStruct((NC, N_ACC, D_IN), jnp.float32),
        jax.ShapeDtypeStruct((NC, N_ACC, 16), jnp.float32),
    ),
    mesh=plsc.VectorSubcoreMesh(core_axis_name="c", subcore_axis_name="s"),
    scratch_types=[
        pltpu.VMEM((CPW, CHUNK), jnp.int32),
        pltpu.VMEM((CPW, CHUNK), jnp.int32),
        pltpu.VMEM((CPW, CHUNK), jnp.int32),
        pltpu.VMEM((CHUNK, D_IN), jnp.float32),
        pltpu.VMEM((CHUNK, 16), jnp.float32),
        pltpu.VMEM_SHARED((N_ACC, D_IN), jnp.float32),
        pltpu.VMEM_SHARED((N_ACC, 16), jnp.float32),
    ],
)
def _sc_aggregate(x_hbm, row_hbm, col_hbm, sums_hbm, cnt_hbm,
                  row_v, col_v, dst_v, msg_v, ones_v, sums_sh, cnt_sh):
    cid = lax.axis_index("c")
    sid = lax.axis_index("s")
    wid = cid * NS + sid

    @pl.loop(0, CHUNK)
    def _zero_rows(r):
        @pl.loop(0, D_IN, step=16)
        def _zero_cols(c):
            msg_v[r, pl.ds(c, 16)] = jnp.zeros((16,), jnp.float32)
        ones_v[r] = jnp.zeros((16,), jnp.float32)

    base = sid * ROWS_PER_SUB

    @pl.loop(0, ROWS_PER_SUB // CHUNK)
    def _zero_sh(k):
        pltpu.sync_copy(msg_v, sums_sh.at[pl.ds(base + k * CHUNK, CHUNK)])
        pltpu.sync_copy(ones_v, cnt_sh.at[pl.ds(base + k * CHUNK, CHUNK)])

    _tail = ROWS_PER_SUB - (ROWS_PER_SUB // CHUNK) * CHUNK
    pltpu.sync_copy(msg_v.at[pl.ds(0, _tail)],
                    sums_sh.at[pl.ds(base + ROWS_PER_SUB - _tail, _tail)])
    pltpu.sync_copy(ones_v.at[pl.ds(0, _tail)],
                    cnt_sh.at[pl.ds(base + ROWS_PER_SUB - _tail, _tail)])

    @pl.loop(0, CHUNK)
    def _fill_ones(r):
        ones_v[r] = jnp.ones((16,), jnp.float32)

    eb = wid * CPW
    pltpu.sync_copy(row_hbm.at[pl.ds(eb, CPW)], row_v)
    pltpu.sync_copy(col_hbm.at[pl.ds(eb, CPW)], col_v)

    dummy = lax.iota(jnp.int32, 16) + N_NODES

    @pl.loop(0, CPW)
    def _dst_chunk(j):
        @pl.loop(0, CHUNK, step=16)
        def _dst_vec(c):
            r = row_v[j, pl.ds(c, 16)]
            s = col_v[j, pl.ds(c, 16)]
            dst_v[j, pl.ds(c, 16)] = jnp.where(r == s, dummy, r)

    plsc.subcore_barrier()

    @pl.loop(0, CPW)
    def _edge_chunk(j):
        pltpu.sync_copy(x_hbm.at[col_v.at[j]], msg_v)
        pltpu.sync_copy(msg_v, sums_sh.at[dst_v.at[j]], add=True)
        pltpu.sync_copy(ones_v, cnt_sh.at[dst_v.at[j]], add=True)

    plsc.subcore_barrier()

    pltpu.sync_copy(sums_sh.at[pl.ds(base, ROWS_PER_SUB)],
                    sums_hbm.at[cid, pl.ds(base, ROWS_PER_SUB)])
    pltpu.sync_copy(cnt_sh.at[pl.ds(base, ROWS_PER_SUB)],
                    cnt_hbm.at[cid, pl.ds(base, ROWS_PER_SUB)])


BLK = 1000


def _mlp_body(x_ref, s0_ref, s1_ref, c0_ref, c1_ref,
              w1_ref, b1_ref, w2_ref, b2_ref, o_ref):
    s = s0_ref[...] + s1_ref[...]
    c = c0_ref[...][:, 0:1] + c1_ref[...][:, 0:1]
    h = x_ref[...] + s / jnp.maximum(c, 1.0)
    h1 = jnp.dot(h, w1_ref[...], preferred_element_type=jnp.float32,
                 precision=lax.Precision.HIGHEST) + b1_ref[...]
    h1 = jnp.maximum(h1, 0.0)
    o_ref[...] = jnp.dot(h1, w2_ref[...], preferred_element_type=jnp.float32,
                         precision=lax.Precision.HIGHEST) + b2_ref[...]


def _tc_mlp(x, s0, s1, c0, c1, W1, b1, W2, b2):
    return pl.pallas_call(
        _mlp_body,
        grid=(N_NODES // BLK,),
        in_specs=[
            pl.BlockSpec((BLK, D_IN), lambda i: (i, 0)),
            pl.BlockSpec((BLK, D_IN), lambda i: (i, 0)),
            pl.BlockSpec((BLK, D_IN), lambda i: (i, 0)),
            pl.BlockSpec((BLK, 16), lambda i: (i, 0)),
            pl.BlockSpec((BLK, 16), lambda i: (i, 0)),
            pl.BlockSpec((D_IN, D_HID), lambda i: (0, 0)),
            pl.BlockSpec((1, D_HID), lambda i: (0, 0)),
            pl.BlockSpec((D_HID, D_OUT), lambda i: (0, 0)),
            pl.BlockSpec((1, D_OUT), lambda i: (0, 0)),
        ],
        out_specs=pl.BlockSpec((BLK, D_OUT), lambda i: (i, 0)),
        out_shape=jax.ShapeDtypeStruct((N_NODES, D_OUT), jnp.float32),
    )(x, s0, s1, c0, c1, W1, b1.reshape(1, D_HID), W2, b2.reshape(1, D_OUT))


def kernel(x, edge_index, W1, b1, W2, b2):
    row = edge_index[0]
    col = edge_index[1]
    pad_n = E_PAD - N_EDGES
    pad_idx = (jnp.arange(pad_n, dtype=jnp.int32) * 97) % N_NODES
    row_p = jnp.concatenate([row, pad_idx]).reshape(NW * CPW, CHUNK)
    col_p = jnp.concatenate([col, pad_idx]).reshape(NW * CPW, CHUNK)

    sums, cnts = _sc_aggregate(x, row_p, col_p)

    return _tc_mlp(
        x,
        sums[0, :N_NODES], sums[1, :N_NODES],
        cnts[0, :N_NODES], cnts[1, :N_NODES],
        W1, b1, W2, b2,
    )


# speedup vs baseline: 4.3691x; 4.3691x over previous
import functools

import jax
import jax.numpy as jnp
from jax import lax
from jax.experimental import pallas as pl
from jax.experimental.pallas import tpu as pltpu
from jax.experimental.pallas import tpu_sc as plsc

N_NODES = 10000
N_EDGES = 320000
D_IN = 128
D_HID = 256
D_OUT = 128

NC = 2
NS = 16
CHUNK = 128
CPS = 160
STG = 32
NSTG = CPS // STG
N_CHUNKS = NS * CPS
E_PAD = N_CHUNKS * CHUNK
N_HALF = 5120
DUM = 2048
N_ACC = N_HALF + DUM
ZERO_PER_SUB = N_ACC // NS
OUT_PER_SUB = N_HALF // NS
N_OUT = NC * N_HALF


@functools.partial(
    pl.kernel,
    out_type=(
        jax.ShapeDtypeStruct((N_OUT, D_IN), jnp.float32),
        jax.ShapeDtypeStruct((N_OUT, D_IN), jnp.float32),
    ),
    mesh=plsc.VectorSubcoreMesh(core_axis_name="c", subcore_axis_name="s"),
    scratch_types=[
        pltpu.VMEM((STG, CHUNK), jnp.int32),
        pltpu.VMEM((STG, CHUNK), jnp.int32),
        pltpu.VMEM((STG, CHUNK), jnp.int32),
        pltpu.VMEM((CHUNK, D_IN), jnp.float32),
        pltpu.VMEM((CHUNK, D_IN), jnp.float32),
        pltpu.VMEM_SHARED((N_ACC, D_IN), jnp.float32),
    ],
)
def _sc_aggregate(x_hbm, row_hbm, col_hbm, zero_hbm, ones_hbm,
                  sums_hbm, cnt_hbm,
                  row_v, col_v, dst_v, msg_v, ones_v, acc_sh):
    cid = lax.axis_index("c")
    sid = lax.axis_index("s")
    lo = cid * N_HALF
    zbase = sid * ZERO_PER_SUB
    obase = sid * OUT_PER_SUB

    def zero_acc():
        pltpu.sync_copy(zero_hbm.at[pl.ds(zbase, ZERO_PER_SUB)],
                        acc_sh.at[pl.ds(zbase, ZERO_PER_SUB)])

    def edge_pass(accumulate):
        @pl.loop(0, NSTG)
        def _stage(st):
            eb = sid * CPS + st * STG
            pltpu.sync_copy(row_hbm.at[pl.ds(eb, STG)], row_v)
            pltpu.sync_copy(col_hbm.at[pl.ds(eb, STG)], col_v)

            @pl.loop(0, STG)
            def _dst_chunk(j):
                @pl.loop(0, CHUNK, step=16)
                def _dst_vec(c):
                    r = row_v[j, pl.ds(c, 16)]
                    s = col_v[j, pl.ds(c, 16)]
                    local = r - lo
                    keep = (r != s) & (local >= 0) & (local < N_HALF)
                    dummy = N_HALF + (s & (DUM - 1))
                    dst_v[j, pl.ds(c, 16)] = jnp.where(keep, local, dummy)

            @pl.loop(0, STG)
            def _edge_chunk(j):
                accumulate(j)

    zero_acc()
    pltpu.sync_copy(ones_hbm, ones_v)
    plsc.subcore_barrier()

    def _sum_edge(j):
        pltpu.sync_copy(x_hbm.at[col_v.at[j]], msg_v)
        pltpu.sync_copy(msg_v, acc_sh.at[dst_v.at[j]], add=True)

    edge_pass(_sum_edge)
    plsc.subcore_barrier()

    pltpu.sync_copy(acc_sh.at[pl.ds(obase, OUT_PER_SUB)],
                    sums_hbm.at[pl.ds(lo + obase, OUT_PER_SUB)])
    plsc.subcore_barrier()

    zero_acc()
    plsc.subcore_barrier()

    def _cnt_edge(j):
        pltpu.sync_copy(ones_v, acc_sh.at[dst_v.at[j]], add=True)

    edge_pass(_cnt_edge)
    plsc.subcore_barrier()

    pltpu.sync_copy(acc_sh.at[pl.ds(obase, OUT_PER_SUB)],
                    cnt_hbm.at[pl.ds(lo + obase, OUT_PER_SUB)])


BLK = 1000


def _mlp_body(x_ref, s_ref, c_ref, w1_ref, b1_ref, w2_ref, b2_ref, o_ref):
    c = c_ref[...][:, 0:1]
    h = x_ref[...] + s_ref[...] / jnp.maximum(c, 1.0)
    h1 = jnp.dot(h, w1_ref[...], preferred_element_type=jnp.float32,
                 precision=lax.Precision.HIGHEST) + b1_ref[...]
    h1 = jnp.maximum(h1, 0.0)
    o_ref[...] = jnp.dot(h1, w2_ref[...], preferred_element_type=jnp.float32,
                         precision=lax.Precision.HIGHEST) + b2_ref[...]


def _tc_mlp(x, s, c, W1, b1, W2, b2):
    return pl.pallas_call(
        _mlp_body,
        grid=(N_NODES // BLK,),
        in_specs=[
            pl.BlockSpec((BLK, D_IN), lambda i: (i, 0)),
            pl.BlockSpec((BLK, D_IN), lambda i: (i, 0)),
            pl.BlockSpec((BLK, D_IN), lambda i: (i, 0)),
            pl.BlockSpec((D_IN, D_HID), lambda i: (0, 0)),
            pl.BlockSpec((1, D_HID), lambda i: (0, 0)),
            pl.BlockSpec((D_HID, D_OUT), lambda i: (0, 0)),
            pl.BlockSpec((1, D_OUT), lambda i: (0, 0)),
        ],
        out_specs=pl.BlockSpec((BLK, D_OUT), lambda i: (i, 0)),
        out_shape=jax.ShapeDtypeStruct((N_NODES, D_OUT), jnp.float32),
    )(x, s, c, W1, b1.reshape(1, D_HID), W2, b2.reshape(1, D_OUT))


def kernel(x, edge_index, W1, b1, W2, b2):
    row = edge_index[0]
    col = edge_index[1]
    pad_n = E_PAD - N_EDGES
    pad_idx = (jnp.arange(pad_n, dtype=jnp.int32) * 97) % N_NODES
    row_p = jnp.concatenate([row, pad_idx]).reshape(N_CHUNKS, CHUNK)
    col_p = jnp.concatenate([col, pad_idx]).reshape(N_CHUNKS, CHUNK)

    zero = jnp.zeros((N_ACC, D_IN), jnp.float32)
    ones = jnp.ones((CHUNK, D_IN), jnp.float32)

    sums, cnts = _sc_aggregate(x, row_p, col_p, zero, ones)

    return _tc_mlp(x, sums[:N_NODES], cnts[:N_NODES], W1, b1, W2, b2)


# speedup vs baseline: 4.8517x; 1.1105x over previous
import functools

import jax
import jax.numpy as jnp
from jax import lax
from jax.experimental import pallas as pl
from jax.experimental.pallas import tpu as pltpu
from jax.experimental.pallas import tpu_sc as plsc

N_NODES = 10000
N_EDGES = 320000
D_IN = 128
D_HID = 256
D_OUT = 128

NC = 2
NS = 16
CHUNK = 128
CPS = 160
STG = 32
NSTG = CPS // STG
N_CHUNKS = NS * CPS
E_PAD = N_CHUNKS * CHUNK
N_HALF = 5120
DUM = 2048
N_ACC = N_HALF + DUM
ZERO_PER_SUB = N_ACC // NS
OUT_PER_SUB = N_HALF // NS
N_OUT = NC * N_HALF


@functools.partial(
    pl.kernel,
    out_type=(
        jax.ShapeDtypeStruct((N_OUT, D_IN), jnp.float32),
        jax.ShapeDtypeStruct((N_OUT, D_IN), jnp.float32),
    ),
    mesh=plsc.VectorSubcoreMesh(core_axis_name="c", subcore_axis_name="s"),
    scratch_types=[
        pltpu.VMEM((STG, CHUNK), jnp.int32),
        pltpu.VMEM((STG, CHUNK), jnp.int32),
        pltpu.VMEM((STG, CHUNK), jnp.int32),
        pltpu.VMEM((CHUNK, D_IN), jnp.float32),
        pltpu.VMEM((CHUNK, D_IN), jnp.float32),
        pltpu.SemaphoreType.DMA,
        pltpu.SemaphoreType.DMA,
        pltpu.VMEM_SHARED((N_ACC, D_IN), jnp.float32),
    ],
)
def _sc_aggregate(x_hbm, row_hbm, col_hbm, zero_hbm, ones_hbm,
                  sums_hbm, cnt_hbm,
                  row_v, col_v, dst_v, msg_a, msg_b, sem_a, sem_b, acc_sh):
    cid = lax.axis_index("c")
    sid = lax.axis_index("s")
    lo = cid * N_HALF
    zbase = sid * ZERO_PER_SUB
    obase = sid * OUT_PER_SUB

    def zero_acc():
        pltpu.sync_copy(zero_hbm.at[pl.ds(zbase, ZERO_PER_SUB)],
                        acc_sh.at[pl.ds(zbase, ZERO_PER_SUB)])

    def edge_pass(accumulate):
        @pl.loop(0, NSTG)
        def _stage(st):
            eb = sid * CPS + st * STG
            pltpu.sync_copy(row_hbm.at[pl.ds(eb, STG)], row_v)
            pltpu.sync_copy(col_hbm.at[pl.ds(eb, STG)], col_v)

            @pl.loop(0, STG)
            def _dst_chunk(j):
                @pl.loop(0, CHUNK, step=16)
                def _dst_vec(c):
                    r = row_v[j, pl.ds(c, 16)]
                    s = col_v[j, pl.ds(c, 16)]
                    local = r - lo
                    keep = (r != s) & (local >= 0) & (local < N_HALF)
                    dummy = N_HALF + (s & (DUM - 1))
                    dst_v[j, pl.ds(c, 16)] = jnp.where(keep, local, dummy)

            @pl.loop(0, STG, step=2)
            def _edge_chunk(j):
                accumulate(j)

    zero_acc()
    plsc.subcore_barrier()

    def _sum_edges(j):
        ga = pltpu.make_async_copy(x_hbm.at[col_v.at[j]], msg_a, sem_a)
        ga.start()
        gb = pltpu.make_async_copy(x_hbm.at[col_v.at[j + 1]], msg_b, sem_b)
        gb.start()
        ga.wait()
        pltpu.sync_copy(msg_a, acc_sh.at[dst_v.at[j]], add=True)
        gb.wait()
        pltpu.sync_copy(msg_b, acc_sh.at[dst_v.at[j + 1]], add=True)

    edge_pass(_sum_edges)
    plsc.subcore_barrier()

    pltpu.sync_copy(acc_sh.at[pl.ds(obase, OUT_PER_SUB)],
                    sums_hbm.at[pl.ds(lo + obase, OUT_PER_SUB)])
    plsc.subcore_barrier()

    zero_acc()
    pltpu.sync_copy(ones_hbm, msg_a)
    plsc.subcore_barrier()

    def _cnt_edges(j):
        pltpu.sync_copy(msg_a, acc_sh.at[dst_v.at[j]], add=True)
        pltpu.sync_copy(msg_a, acc_sh.at[dst_v.at[j + 1]], add=True)

    edge_pass(_cnt_edges)
    plsc.subcore_barrier()

    pltpu.sync_copy(acc_sh.at[pl.ds(obase, OUT_PER_SUB)],
                    cnt_hbm.at[pl.ds(lo + obase, OUT_PER_SUB)])


BLK = 1000


def _mlp_body(x_ref, s_ref, c_ref, w1_ref, b1_ref, w2_ref, b2_ref, o_ref):
    c = c_ref[...][:, 0:1]
    h = x_ref[...] + s_ref[...] / jnp.maximum(c, 1.0)
    h1 = jnp.dot(h, w1_ref[...], preferred_element_type=jnp.float32,
                 precision=lax.Precision.HIGHEST) + b1_ref[...]
    h1 = jnp.maximum(h1, 0.0)
    o_ref[...] = jnp.dot(h1, w2_ref[...], preferred_element_type=jnp.float32,
                         precision=lax.Precision.HIGHEST) + b2_ref[...]


def _tc_mlp(x, s, c, W1, b1, W2, b2):
    return pl.pallas_call(
        _mlp_body,
        grid=(N_NODES // BLK,),
        in_specs=[
            pl.BlockSpec((BLK, D_IN), lambda i: (i, 0)),
            pl.BlockSpec((BLK, D_IN), lambda i: (i, 0)),
            pl.BlockSpec((BLK, D_IN), lambda i: (i, 0)),
            pl.BlockSpec((D_IN, D_HID), lambda i: (0, 0)),
            pl.BlockSpec((1, D_HID), lambda i: (0, 0)),
            pl.BlockSpec((D_HID, D_OUT), lambda i: (0, 0)),
            pl.BlockSpec((1, D_OUT), lambda i: (0, 0)),
        ],
        out_specs=pl.BlockSpec((BLK, D_OUT), lambda i: (i, 0)),
        out_shape=jax.ShapeDtypeStruct((N_NODES, D_OUT), jnp.float32),
    )(x, s, c, W1, b1.reshape(1, D_HID), W2, b2.reshape(1, D_OUT))


def kernel(x, edge_index, W1, b1, W2, b2):
    row = edge_index[0]
    col = edge_index[1]
    pad_n = E_PAD - N_EDGES
    pad_idx = (jnp.arange(pad_n, dtype=jnp.int32) * 97) % N_NODES
    row_p = jnp.concatenate([row, pad_idx]).reshape(N_CHUNKS, CHUNK)
    col_p = jnp.concatenate([col, pad_idx]).reshape(N_CHUNKS, CHUNK)

    zero = jnp.zeros((N_ACC, D_IN), jnp.float32)
    ones = jnp.ones((CHUNK, D_IN), jnp.float32)

    sums, cnts = _sc_aggregate(x, row_p, col_p, zero, ones)

    return _tc_mlp(x, sums[:N_NODES], cnts[:N_NODES], W1, b1, W2, b2)
